# baseline (speedup 1.0000x reference)
import functools

import jax
import jax.numpy as jnp
from jax import lax
from jax.experimental import pallas as pl
from jax.experimental.pallas import tpu as pltpu
from jax.experimental.pallas import tpu_sc as plsc

VOCAB = 1000000
HIDDEN = 64
B = 16384
L = 50

NUM_CORES = 2
NUM_SUBCORES = 16
NW = NUM_CORES * NUM_SUBCORES
ROWS_W = B // NW
NB = 2
RB = 16
STEPS = ROWS_W // RB
MAIN = (STEPS - NB) // NB

_mesh = plsc.VectorSubcoreMesh(core_axis_name="c", subcore_axis_name="s")


@functools.partial(
    pl.kernel,
    mesh=_mesh,
    out_type=jax.ShapeDtypeStruct((B, L, HIDDEN), jnp.float32),
    scratch_types=[
        pltpu.VMEM((RB, L), jnp.int32),
        pltpu.VMEM((RB, L), jnp.int32),
        pltpu.VMEM((RB, L, HIDDEN), jnp.float32),
        pltpu.VMEM((RB, L, HIDDEN), jnp.float32),
        pltpu.SemaphoreType.DMA,
        pltpu.SemaphoreType.DMA,
        pltpu.SemaphoreType.DMA,
        pltpu.SemaphoreType.DMA,
        pltpu.SemaphoreType.DMA,
        pltpu.SemaphoreType.DMA,
    ],
    compiler_params=pltpu.CompilerParams(use_tc_tiling_on_sc=False),
)
def _gather_kernel(x_hbm, table_hbm, out_hbm,
                   idx0, idx1, rows0, rows1,
                   isem0, isem1, gsem0, gsem1, ssem0, ssem1):
    idx_v = (idx0, idx1)
    rows_v = (rows0, rows1)
    isem = (isem0, isem1)
    gsem = (gsem0, gsem1)
    ssem = (ssem0, ssem1)

    wid = lax.axis_index("s") * NUM_CORES + lax.axis_index("c")
    base = wid * ROWS_W

    def row0(i):
        return pl.multiple_of(base + i * RB, 8)

    def start_gathers(b):
        for j in range(RB):
            pltpu.async_copy(table_hbm.at[idx_v[b].at[j]], rows_v[b].at[j], gsem[b])

    def wait_gathers(b):
        for j in range(RB):
            pltpu.make_async_copy(table_hbm.at[idx_v[b].at[j]], rows_v[b].at[j],
                                  gsem[b]).wait()

    for b in range(NB):
        pltpu.async_copy(x_hbm.at[pl.ds(row0(b), RB), :], idx_v[b], isem[b]).wait()
        start_gathers(b)

    def body(g, carry):
        for b in range(NB):
            i = g * NB + b
            wait_gathers(b)
            dstore = pltpu.async_copy(rows_v[b], out_hbm.at[pl.ds(row0(i), RB)], ssem[b])
            didx = pltpu.async_copy(x_hbm.at[pl.ds(row0(i + NB), RB), :], idx_v[b], isem[b])
            didx.wait()
            dstore.wait()
            start_gathers(b)
        return carry

    lax.fori_loop(0, MAIN, body, 0)

    for b in range(NB):
        i = STEPS - NB + b
        wait_gathers(b)
        pltpu.async_copy(rows_v[b], out_hbm.at[pl.ds(row0(i), RB)], ssem[b]).wait()


def kernel(x, table):
    return _gather_kernel(x.astype(jnp.int32), table)
